# Initial kernel scaffold; baseline (speedup 1.0000x reference)
import jax
import jax.numpy as jnp
from jax.experimental import pallas as pl


def kernel(features, edge_index, W1, b1, W2, b2):
    raise NotImplementedError("write your pallas kernel here")


# speedup vs baseline: 4.3466x; 4.3466x over previous
import functools

import jax
import jax.numpy as jnp
from jax import lax
from jax.experimental import pallas as pl
from jax.experimental.pallas import tpu as pltpu
from jax.experimental.pallas import tpu_sc as plsc

N = 10000
NP = 10240
E = 320000
D_IN = 128
D_H = 128
D_OUT = 64

NC = 2
NS = 16
NW = NC * NS
EPW = E // NW
K = 80
NCHUNK = EPW // K
RPT = NP // NS


def _sc_mesh():
    return plsc.VectorSubcoreMesh(core_axis_name="c", subcore_axis_name="s")


_SC_PARAMS = pltpu.CompilerParams(use_tc_tiling_on_sc=False)


def _make_deg_kernel():
    @functools.partial(
        pl.kernel,
        out_type=(
            jax.ShapeDtypeStruct((NC, NP), jnp.float32),
            jax.ShapeDtypeStruct((NC, NP), jnp.float32),
        ),
        mesh=_sc_mesh(),
        scratch_types=[
            pltpu.VMEM((K,), jnp.int32),
            pltpu.VMEM((K,), jnp.float32),
            pltpu.VMEM_SHARED((NP,), jnp.float32),
            pltpu.VMEM_SHARED((NP,), jnp.float32),
        ],
        compiler_params=_SC_PARAMS,
    )
    def deg_kernel(src_hbm, dst_hbm, zeros_hbm, dout_hbm, din_hbm,
                   idx_v, ones_v, dsrc_sh, ddst_sh):
        c = lax.axis_index("c")
        s = lax.axis_index("s")
        wid = c * NS + s
        base_n = s * RPT
        pltpu.sync_copy(zeros_hbm.at[pl.ds(base_n, RPT)],
                        dsrc_sh.at[pl.ds(base_n, RPT)])
        pltpu.sync_copy(zeros_hbm.at[pl.ds(base_n, RPT)],
                        ddst_sh.at[pl.ds(base_n, RPT)])
        for i in range(K // 16):
            ones_v[pl.ds(i * 16, 16)] = jnp.ones((16,), jnp.float32)
        plsc.subcore_barrier()

        def body(j, carry):
            base_e = wid * EPW + j * K
            pltpu.sync_copy(src_hbm.at[pl.ds(base_e, K)], idx_v)
            pltpu.sync_copy(ones_v, dsrc_sh.at[idx_v], add=True)
            pltpu.sync_copy(dst_hbm.at[pl.ds(base_e, K)], idx_v)
            pltpu.sync_copy(ones_v, ddst_sh.at[idx_v], add=True)
            return carry

        lax.fori_loop(0, NCHUNK, body, 0)
        plsc.subcore_barrier()
        pltpu.sync_copy(dsrc_sh.at[pl.ds(base_n, RPT)],
                        dout_hbm.at[c, pl.ds(base_n, RPT)])
        pltpu.sync_copy(ddst_sh.at[pl.ds(base_n, RPT)],
                        din_hbm.at[c, pl.ds(base_n, RPT)])

    return deg_kernel


def _make_agg_kernel(d):
    @functools.partial(
        pl.kernel,
        out_type=jax.ShapeDtypeStruct((NC, NP, d), jnp.float32),
        mesh=_sc_mesh(),
        scratch_types=[
            pltpu.VMEM((K,), jnp.int32),
            pltpu.VMEM((K,), jnp.int32),
            pltpu.VMEM((K, d), jnp.float32),
            pltpu.VMEM_SHARED((NP, d), jnp.float32),
            pltpu.SemaphoreType.DMA,
        ],
        compiler_params=_SC_PARAMS,
    )
    def agg_kernel(src_hbm, dst_hbm, h_hbm, zeros_hbm, out_hbm,
                   sidx_v, didx_v, rows_v, agg_sh, sem):
        c = lax.axis_index("c")
        s = lax.axis_index("s")
        wid = c * NS + s
        base_n = s * RPT
        pltpu.sync_copy(zeros_hbm.at[pl.ds(base_n, RPT)],
                        agg_sh.at[pl.ds(base_n, RPT)])
        plsc.subcore_barrier()

        def body(j, carry):
            base_e = wid * EPW + j * K
            pltpu.sync_copy(src_hbm.at[pl.ds(base_e, K)], sidx_v)
            pltpu.sync_copy(dst_hbm.at[pl.ds(base_e, K)], didx_v)
            pltpu.async_copy(h_hbm.at[sidx_v], rows_v, sem).wait()
            pltpu.sync_copy(rows_v, agg_sh.at[didx_v], add=True)
            return carry

        lax.fori_loop(0, NCHUNK, body, 0)
        plsc.subcore_barrier()
        pltpu.sync_copy(agg_sh.at[pl.ds(base_n, RPT)],
                        out_hbm.at[c, pl.ds(base_n, RPT)])

    return agg_kernel


BLK = 2048


def _layer1_body(dout_ref, din_ref, x_ref, w_ref, so_ref, si_ref, h_ref):
    deg_out = jnp.maximum(dout_ref[0, :] + dout_ref[1, :], 1.0)
    deg_in = jnp.maximum(din_ref[0, :] + din_ref[1, :], 1.0)
    so = lax.rsqrt(deg_out)
    si = lax.rsqrt(deg_in)
    so_ref[...] = so
    si_ref[...] = si
    h_ref[...] = jnp.dot(x_ref[...] * so[:, None], w_ref[...],
                         preferred_element_type=jnp.float32)


def _tc_layer1(dout_p, din_p, x_pad, W1):
    return pl.pallas_call(
        _layer1_body,
        grid=(NP // BLK,),
        in_specs=[
            pl.BlockSpec((NC, BLK), lambda i: (0, i)),
            pl.BlockSpec((NC, BLK), lambda i: (0, i)),
            pl.BlockSpec((BLK, D_IN), lambda i: (i, 0)),
            pl.BlockSpec((D_IN, D_H), lambda i: (0, 0)),
        ],
        out_specs=[
            pl.BlockSpec((BLK,), lambda i: (i,)),
            pl.BlockSpec((BLK,), lambda i: (i,)),
            pl.BlockSpec((BLK, D_H), lambda i: (i, 0)),
        ],
        out_shape=[
            jax.ShapeDtypeStruct((NP,), jnp.float32),
            jax.ShapeDtypeStruct((NP,), jnp.float32),
            jax.ShapeDtypeStruct((NP, D_H), jnp.float32),
        ],
    )(dout_p, din_p, x_pad, W1)


def _layer2_body(agg_ref, si_ref, so_ref, b_ref, w_ref, h_ref):
    agg = agg_ref[0, :, :] + agg_ref[1, :, :]
    h = agg * si_ref[...][:, None] + b_ref[...][None, :]
    h = jnp.maximum(h, 0.0)
    h_ref[...] = jnp.dot(h * so_ref[...][:, None], w_ref[...],
                         preferred_element_type=jnp.float32)


def _tc_layer2(aggp1, si, so, b1, W2):
    return pl.pallas_call(
        _layer2_body,
        grid=(NP // BLK,),
        in_specs=[
            pl.BlockSpec((NC, BLK, D_H), lambda i: (0, i, 0)),
            pl.BlockSpec((BLK,), lambda i: (i,)),
            pl.BlockSpec((BLK,), lambda i: (i,)),
            pl.BlockSpec((D_H,), lambda i: (0,)),
            pl.BlockSpec((D_H, D_OUT), lambda i: (0, 0)),
        ],
        out_specs=pl.BlockSpec((BLK, D_OUT), lambda i: (i, 0)),
        out_shape=jax.ShapeDtypeStruct((NP, D_OUT), jnp.float32),
    )(aggp1, si, so, b1, W2)


def _final_body(agg_ref, si_ref, b_ref, out_ref):
    agg = agg_ref[0, :, :] + agg_ref[1, :, :]
    out_ref[...] = agg * si_ref[...][:, None] + b_ref[...][None, :]


def _tc_final(aggp2, si, b2):
    return pl.pallas_call(
        _final_body,
        grid=(NP // BLK,),
        in_specs=[
            pl.BlockSpec((NC, BLK, D_OUT), lambda i: (0, i, 0)),
            pl.BlockSpec((BLK,), lambda i: (i,)),
            pl.BlockSpec((D_OUT,), lambda i: (0,)),
        ],
        out_specs=pl.BlockSpec((BLK, D_OUT), lambda i: (i, 0)),
        out_shape=jax.ShapeDtypeStruct((NP, D_OUT), jnp.float32),
    )(aggp2, si, b2)


def kernel(features, edge_index, W1, b1, W2, b2):
    src = edge_index[0]
    dst = edge_index[1]
    x_pad = jnp.pad(features, ((0, NP - N), (0, 0)))
    zeros_n = jnp.zeros((NP,), jnp.float32)
    zeros_h = jnp.zeros((NP, D_H), jnp.float32)
    zeros_o = jnp.zeros((NP, D_OUT), jnp.float32)

    dout_p, din_p = _make_deg_kernel()(src, dst, zeros_n)
    so, si, h1 = _tc_layer1(dout_p, din_p, x_pad, W1)
    aggp1 = _make_agg_kernel(D_H)(src, dst, h1, zeros_h)
    h2 = _tc_layer2(aggp1, si, so, b1, W2)
    aggp2 = _make_agg_kernel(D_OUT)(src, dst, h2, zeros_o)
    return _tc_final(aggp2, si, b2)[:N]


# speedup vs baseline: 8.6822x; 1.9975x over previous
import functools

import jax
import jax.numpy as jnp
from jax import lax
from jax.experimental import pallas as pl
from jax.experimental.pallas import tpu as pltpu
from jax.experimental.pallas import tpu_sc as plsc

N = 10000
NP = 10240
E = 320000
D_IN = 128
D_H = 128
D_OUT = 64

NC = 2
NS = 16
NW = NC * NS
EPW = E // NW
K = 80
NCHUNK = EPW // K
RPT = NP // NS


def _sc_mesh():
    return plsc.VectorSubcoreMesh(core_axis_name="c", subcore_axis_name="s")


_SC_PARAMS = pltpu.CompilerParams(use_tc_tiling_on_sc=False)


NB = 3
_MAIN = (NCHUNK // NB) * NB


def _make_deg_kernel():
    @functools.partial(
        pl.kernel,
        out_type=(
            jax.ShapeDtypeStruct((NC, NP), jnp.float32),
            jax.ShapeDtypeStruct((NC, NP), jnp.float32),
        ),
        mesh=_sc_mesh(),
        scratch_types=(
            [pltpu.VMEM((K,), jnp.int32) for _ in range(NB)]
            + [pltpu.VMEM((K,), jnp.int32) for _ in range(NB)]
            + [pltpu.VMEM((K,), jnp.float32),
               pltpu.VMEM_SHARED((NP,), jnp.float32),
               pltpu.VMEM_SHARED((NP,), jnp.float32)]
            + [pltpu.SemaphoreType.DMA for _ in range(NB)]
            + [pltpu.SemaphoreType.DMA]
        ),
        compiler_params=_SC_PARAMS,
    )
    def deg_kernel(src_hbm, dst_hbm, zeros_hbm, dout_hbm, din_hbm, *refs):
        sidx = refs[0:NB]
        didx = refs[NB:2 * NB]
        ones_v, dsrc_sh, ddst_sh = refs[2 * NB:2 * NB + 3]
        isem = refs[2 * NB + 3:2 * NB + 3 + NB]
        ssem = refs[2 * NB + 3 + NB]
        c = lax.axis_index("c")
        s = lax.axis_index("s")
        wid = c * NS + s
        base_n = s * RPT
        pltpu.sync_copy(zeros_hbm.at[pl.ds(base_n, RPT)],
                        dsrc_sh.at[pl.ds(base_n, RPT)])
        pltpu.sync_copy(zeros_hbm.at[pl.ds(base_n, RPT)],
                        ddst_sh.at[pl.ds(base_n, RPT)])
        for i in range(K // 16):
            ones_v[pl.ds(i * 16, 16)] = jnp.ones((16,), jnp.float32)
        plsc.subcore_barrier()

        def start_idx(j, m):
            base_e = wid * EPW + jnp.minimum(j, NCHUNK - 1) * K
            pltpu.async_copy(src_hbm.at[pl.ds(base_e, K)], sidx[m], isem[m])
            pltpu.async_copy(dst_hbm.at[pl.ds(base_e, K)], didx[m], isem[m])

        def wait_idx(m):
            pltpu.make_async_copy(src_hbm.at[pl.ds(0, K)], sidx[m],
                                  isem[m]).wait()
            pltpu.make_async_copy(dst_hbm.at[pl.ds(0, K)], didx[m],
                                  isem[m]).wait()

        def scatter(p):
            pltpu.async_copy(ones_v, dsrc_sh.at[sidx[p]], ssem, add=True)
            pltpu.sync_copy(ones_v, ddst_sh.at[didx[p]], add=True)
            pltpu.make_async_copy(zeros_hbm.at[pl.ds(0, K)], ones_v,
                                  ssem).wait()

        start_idx(0, 0)
        start_idx(1, 1)

        def body(g, carry):
            for p in range(NB):
                j = g * NB + p
                wait_idx(p)
                start_idx(j + 2, (p + 2) % NB)
                scatter(p)
            return carry

        lax.fori_loop(0, _MAIN // NB, body, 0)
        if _MAIN == NCHUNK:
            wait_idx(NCHUNK % NB)
            wait_idx((NCHUNK + 1) % NB)
        else:
            for j in range(_MAIN, NCHUNK):
                wait_idx(j % NB)
                scatter(j % NB)
        plsc.subcore_barrier()
        pltpu.sync_copy(dsrc_sh.at[pl.ds(base_n, RPT)],
                        dout_hbm.at[c, pl.ds(base_n, RPT)])
        pltpu.sync_copy(ddst_sh.at[pl.ds(base_n, RPT)],
                        din_hbm.at[c, pl.ds(base_n, RPT)])

    return deg_kernel


def _make_agg_kernel(d):
    @functools.partial(
        pl.kernel,
        out_type=jax.ShapeDtypeStruct((NC, NP, d), jnp.float32),
        mesh=_sc_mesh(),
        scratch_types=(
            [pltpu.VMEM((K,), jnp.int32) for _ in range(NB)]
            + [pltpu.VMEM((K,), jnp.int32) for _ in range(NB)]
            + [pltpu.VMEM((K, d), jnp.float32) for _ in range(NB)]
            + [pltpu.VMEM_SHARED((NP, d), jnp.float32)]
            + [pltpu.SemaphoreType.DMA for _ in range(2 * NB)]
        ),
        compiler_params=_SC_PARAMS,
    )
    def agg_kernel(src_hbm, dst_hbm, h_hbm, zeros_hbm, out_hbm, *refs):
        sidx = refs[0:NB]
        didx = refs[NB:2 * NB]
        rows = refs[2 * NB:3 * NB]
        agg_sh = refs[3 * NB]
        isem = refs[3 * NB + 1:3 * NB + 1 + NB]
        gsem = refs[3 * NB + 1 + NB:3 * NB + 1 + 2 * NB]
        c = lax.axis_index("c")
        s = lax.axis_index("s")
        wid = c * NS + s
        base_n = s * RPT
        pltpu.sync_copy(zeros_hbm.at[pl.ds(base_n, RPT)],
                        agg_sh.at[pl.ds(base_n, RPT)])

        def start_idx(j, m):
            base_e = wid * EPW + jnp.minimum(j, NCHUNK - 1) * K
            pltpu.async_copy(src_hbm.at[pl.ds(base_e, K)], sidx[m], isem[m])
            pltpu.async_copy(dst_hbm.at[pl.ds(base_e, K)], didx[m], isem[m])

        def wait_idx(m):
            pltpu.make_async_copy(src_hbm.at[pl.ds(0, K)], sidx[m],
                                  isem[m]).wait()
            pltpu.make_async_copy(dst_hbm.at[pl.ds(0, K)], didx[m],
                                  isem[m]).wait()

        def start_gather(m):
            pltpu.async_copy(h_hbm.at[sidx[m]], rows[m], gsem[m])

        def wait_gather(m):
            pltpu.make_async_copy(h_hbm.at[pl.ds(0, K)], rows[m],
                                  gsem[m]).wait()

        plsc.subcore_barrier()
        start_idx(0, 0)
        start_idx(1, 1)
        wait_idx(0)
        start_gather(0)

        def body(g, carry):
            for p in range(NB):
                j = g * NB + p
                p1 = (p + 1) % NB
                wait_gather(p)
                wait_idx(p1)
                start_gather(p1)
                start_idx(j + 2, (p + 2) % NB)
                pltpu.sync_copy(rows[p], agg_sh.at[didx[p]], add=True)
            return carry

        lax.fori_loop(0, _MAIN // NB, body, 0)
        if _MAIN == NCHUNK:
            wait_gather(NCHUNK % NB)
            wait_idx((NCHUNK + 1) % NB)
        else:
            for j in range(_MAIN, NCHUNK):
                p = j % NB
                wait_gather(p)
                if j + 1 < NCHUNK:
                    p1 = (j + 1) % NB
                    wait_idx(p1)
                    start_gather(p1)
                pltpu.sync_copy(rows[p], agg_sh.at[didx[p]], add=True)
        plsc.subcore_barrier()
        pltpu.sync_copy(agg_sh.at[pl.ds(base_n, RPT)],
                        out_hbm.at[c, pl.ds(base_n, RPT)])

    return agg_kernel


BLK = 2048


def _layer1_body(dout_ref, din_ref, x_ref, w_ref, so_ref, si_ref, h_ref):
    deg_out = jnp.maximum(dout_ref[0, :] + dout_ref[1, :], 1.0)
    deg_in = jnp.maximum(din_ref[0, :] + din_ref[1, :], 1.0)
    so = lax.rsqrt(deg_out)
    si = lax.rsqrt(deg_in)
    so_ref[...] = so
    si_ref[...] = si
    h_ref[...] = jnp.dot(x_ref[...] * so[:, None], w_ref[...],
                         preferred_element_type=jnp.float32)


def _tc_layer1(dout_p, din_p, x_pad, W1):
    return pl.pallas_call(
        _layer1_body,
        grid=(NP // BLK,),
        in_specs=[
            pl.BlockSpec((NC, BLK), lambda i: (0, i)),
            pl.BlockSpec((NC, BLK), lambda i: (0, i)),
            pl.BlockSpec((BLK, D_IN), lambda i: (i, 0)),
            pl.BlockSpec((D_IN, D_H), lambda i: (0, 0)),
        ],
        out_specs=[
            pl.BlockSpec((BLK,), lambda i: (i,)),
            pl.BlockSpec((BLK,), lambda i: (i,)),
            pl.BlockSpec((BLK, D_H), lambda i: (i, 0)),
        ],
        out_shape=[
            jax.ShapeDtypeStruct((NP,), jnp.float32),
            jax.ShapeDtypeStruct((NP,), jnp.float32),
            jax.ShapeDtypeStruct((NP, D_H), jnp.float32),
        ],
    )(dout_p, din_p, x_pad, W1)


def _layer2_body(agg_ref, si_ref, so_ref, b_ref, w_ref, h_ref):
    agg = agg_ref[0, :, :] + agg_ref[1, :, :]
    h = agg * si_ref[...][:, None] + b_ref[...][None, :]
    h = jnp.maximum(h, 0.0)
    h_ref[...] = jnp.dot(h * so_ref[...][:, None], w_ref[...],
                         preferred_element_type=jnp.float32)


def _tc_layer2(aggp1, si, so, b1, W2):
    return pl.pallas_call(
        _layer2_body,
        grid=(NP // BLK,),
        in_specs=[
            pl.BlockSpec((NC, BLK, D_H), lambda i: (0, i, 0)),
            pl.BlockSpec((BLK,), lambda i: (i,)),
            pl.BlockSpec((BLK,), lambda i: (i,)),
            pl.BlockSpec((D_H,), lambda i: (0,)),
            pl.BlockSpec((D_H, D_OUT), lambda i: (0, 0)),
        ],
        out_specs=pl.BlockSpec((BLK, D_OUT), lambda i: (i, 0)),
        out_shape=jax.ShapeDtypeStruct((NP, D_OUT), jnp.float32),
    )(aggp1, si, so, b1, W2)


def _final_body(agg_ref, si_ref, b_ref, out_ref):
    agg = agg_ref[0, :, :] + agg_ref[1, :, :]
    out_ref[...] = agg * si_ref[...][:, None] + b_ref[...][None, :]


def _tc_final(aggp2, si, b2):
    return pl.pallas_call(
        _final_body,
        grid=(NP // BLK,),
        in_specs=[
            pl.BlockSpec((NC, BLK, D_OUT), lambda i: (0, i, 0)),
            pl.BlockSpec((BLK,), lambda i: (i,)),
            pl.BlockSpec((D_OUT,), lambda i: (0,)),
        ],
        out_specs=pl.BlockSpec((BLK, D_OUT), lambda i: (i, 0)),
        out_shape=jax.ShapeDtypeStruct((NP, D_OUT), jnp.float32),
    )(aggp2, si, b2)


def kernel(features, edge_index, W1, b1, W2, b2):
    src = edge_index[0]
    dst = edge_index[1]
    x_pad = jnp.pad(features, ((0, NP - N), (0, 0)))
    zeros_n = jnp.zeros((NP,), jnp.float32)
    zeros_h = jnp.zeros((NP, D_H), jnp.float32)
    zeros_o = jnp.zeros((NP, D_OUT), jnp.float32)

    dout_p, din_p = _make_deg_kernel()(src, dst, zeros_n)
    so, si, h1 = _tc_layer1(dout_p, din_p, x_pad, W1)
    aggp1 = _make_agg_kernel(D_H)(src, dst, h1, zeros_h)
    h2 = _tc_layer2(aggp1, si, so, b1, W2)
    aggp2 = _make_agg_kernel(D_OUT)(src, dst, h2, zeros_o)
    return _tc_final(aggp2, si, b2)[:N]
